# baseline (speedup 1.0000x reference)
import jax
import jax.numpy as jnp
from jax.experimental import pallas as pl
from jax.experimental.pallas import tpu as pltpu

M = 64
S = 12
R = 3
D = 1024
T = M * R * S

TRIPLES_PER_TILE = 32
TILE_T = TRIPLES_PER_TILE * R * S


def _body(x_ref, wt_ref, wr_ref, wk_ref, lat_ref, cp_ref):
    x = x_ref[...]
    wt = wt_ref[...]
    wr = wr_ref[...]
    wk = wk_ref[...]
    p36 = (jnp.repeat(wr, S, axis=0) + jnp.tile(wk, (R, 1)))
    pos = (wt[:, None, :] + p36[None, :, :]).reshape(TILE_T, D)
    lat_ref[...] = x + pos[None]
    cp_ref[...] = x


def kernel(token_embeds, pad_mask, W_triple, W_role, W_tokpos):
    B = token_embeds.shape[0]
    grid = (B, T // TILE_T)
    out_sds = jax.ShapeDtypeStruct((B, T, D), token_embeds.dtype)
    latent, copy = pl.pallas_call(
        _body,
        grid=grid,
        in_specs=[
            pl.BlockSpec((1, TILE_T, D), lambda b, t: (b, t, 0)),
            pl.BlockSpec((TRIPLES_PER_TILE, D), lambda b, t: (t, 0)),
            pl.BlockSpec((R, D), lambda b, t: (0, 0)),
            pl.BlockSpec((S, D), lambda b, t: (0, 0)),
        ],
        out_specs=[
            pl.BlockSpec((1, TILE_T, D), lambda b, t: (b, t, 0)),
            pl.BlockSpec((1, TILE_T, D), lambda b, t: (b, t, 0)),
        ],
        out_shape=[out_sds, out_sds],
        compiler_params=pltpu.CompilerParams(
            dimension_semantics=("parallel", "parallel"),
        ),
    )(token_embeds, W_triple, W_role, W_tokpos)
    return (latent, copy)


# speedup vs baseline: 1.0091x; 1.0091x over previous
import jax
import jax.numpy as jnp
from jax.experimental import pallas as pl
from jax.experimental.pallas import tpu as pltpu

M = 64
S = 12
R = 3
D = 1024
T = M * R * S

TRIPLES_PER_TILE = 8
TILE_T = TRIPLES_PER_TILE * R * S


def _body(x_ref, wt_ref, wr_ref, wk_ref, lat_ref, cp_ref):
    x = x_ref[...]
    wt = wt_ref[...]
    wr = wr_ref[...]
    wk = wk_ref[...]
    p36 = (jnp.repeat(wr, S, axis=0) + jnp.tile(wk, (R, 1)))
    pos = (wt[:, None, :] + p36[None, :, :]).reshape(TILE_T, D)
    lat_ref[...] = x + pos[None]
    cp_ref[...] = x


def kernel(token_embeds, pad_mask, W_triple, W_role, W_tokpos):
    B = token_embeds.shape[0]
    grid = (T // TILE_T,)
    out_sds = jax.ShapeDtypeStruct((B, T, D), token_embeds.dtype)
    latent, copy = pl.pallas_call(
        _body,
        grid=grid,
        in_specs=[
            pl.BlockSpec((B, TILE_T, D), lambda t: (0, t, 0)),
            pl.BlockSpec((TRIPLES_PER_TILE, D), lambda t: (t, 0)),
            pl.BlockSpec((R, D), lambda t: (0, 0)),
            pl.BlockSpec((S, D), lambda t: (0, 0)),
        ],
        out_specs=[
            pl.BlockSpec((B, TILE_T, D), lambda t: (0, t, 0)),
            pl.BlockSpec((B, TILE_T, D), lambda t: (0, t, 0)),
        ],
        out_shape=[out_sds, out_sds],
        compiler_params=pltpu.CompilerParams(
            dimension_semantics=("parallel",),
        ),
    )(token_embeds, W_triple, W_role, W_tokpos)
    return (latent, copy)
